# Initial kernel scaffold; baseline (speedup 1.0000x reference)
import jax
import jax.numpy as jnp
from jax.experimental import pallas as pl


def kernel(coords, emb0, emb1, emb2, emb3):
    raise NotImplementedError("write your pallas kernel here")


# speedup vs baseline: 4.7117x; 4.7117x over previous
import functools

import jax
import jax.numpy as jnp
from jax import lax
from jax.experimental import pallas as pl
from jax.experimental.pallas import tpu as pltpu
from jax.experimental.pallas import tpu_sc as plsc

N = 16384
D = 128
LEVELS = 4
NC = 2
NS = 16
NW = NC * NS
ROWS_PER_W = N // NW
CHUNK = 128
NCHUNK = ROWS_PER_W // CHUNK
LANES = 16


def _body(coords_r, e0, e1, e2, e3, out, idx_v, a_v, b_v, c_v, d_v, sem):
    wid = lax.axis_index("s") * NC + lax.axis_index("c")
    base = wid * ROWS_PER_W
    pltpu.sync_copy(coords_r.at[wid], idx_v)
    tables = (e0, e1, e2, e3)
    bufs = (a_v, b_v, c_v, d_v)
    for k in range(NCHUNK):
        cps = [
            pltpu.async_copy(tables[l].at[idx_v.at[l, k]], bufs[l], sem)
            for l in range(LEVELS)
        ]
        for cp in cps:
            cp.wait()

        def add_row(r, _):
            for col in range(D // LANES):
                sl = pl.ds(col * LANES, LANES)
                a_v[r, sl] = a_v[r, sl] + b_v[r, sl] + c_v[r, sl] + d_v[r, sl]
            return 0

        lax.fori_loop(0, CHUNK, add_row, 0)
        pltpu.sync_copy(a_v, out.at[pl.ds(base + k * CHUNK, CHUNK)])


_mesh = plsc.VectorSubcoreMesh(core_axis_name="c", subcore_axis_name="s")

_sc_call = functools.partial(
    pl.kernel,
    mesh=_mesh,
    out_type=jax.ShapeDtypeStruct((N, D), jnp.float32),
    scratch_types=[
        pltpu.VMEM((LEVELS, NCHUNK, CHUNK), jnp.int32),
        pltpu.VMEM((CHUNK, D), jnp.float32),
        pltpu.VMEM((CHUNK, D), jnp.float32),
        pltpu.VMEM((CHUNK, D), jnp.float32),
        pltpu.VMEM((CHUNK, D), jnp.float32),
        pltpu.SemaphoreType.DMA,
    ],
)(_body)


@jax.jit
def kernel(coords, emb0, emb1, emb2, emb3):
    coords_r = coords.T.reshape(LEVELS, NW, NCHUNK, CHUNK).transpose(1, 0, 2, 3)
    return _sc_call(coords_r, emb0, emb1, emb2, emb3)


# speedup vs baseline: 5.3605x; 1.1377x over previous
import functools

import jax
import jax.numpy as jnp
from jax import lax
from jax.experimental import pallas as pl
from jax.experimental.pallas import tpu as pltpu
from jax.experimental.pallas import tpu_sc as plsc

N = 16384
D = 128
LEVELS = 4
NC = 2
NS = 16
NW = NC * NS
ROWS_PER_W = N // NW
CHUNK = 64
NCHUNK = ROWS_PER_W // CHUNK
LANES = 16


def _body(coords_r, e0, e1, e2, e3, out, idx_v,
          g0, g1, g2, g3, h0, h1, h2, h3, o0, o1,
          gs0, gs1, ws0, ws1):
    wid = lax.axis_index("s") * NC + lax.axis_index("c")
    base = wid * ROWS_PER_W
    pltpu.sync_copy(coords_r.at[wid], idx_v)
    tables = (e0, e1, e2, e3)
    bufsets = ((g0, g1, g2, g3), (h0, h1, h2, h3))
    obufs = (o0, o1)
    gsems = (gs0, gs1)
    wsems = (ws0, ws1)

    def fire_gathers(k, par):
        return [
            pltpu.async_copy(tables[l].at[idx_v.at[l, k]], bufsets[par][l],
                             gsems[par])
            for l in range(LEVELS)
        ]

    gcps = [fire_gathers(0, 0), None]
    wcps = [None, None]
    for k in range(NCHUNK):
        cur, nxt = k % 2, (k + 1) % 2
        if k + 1 < NCHUNK:
            gcps[nxt] = fire_gathers(k + 1, nxt)
        for cp in gcps[cur]:
            cp.wait()
        if wcps[cur] is not None:
            wcps[cur].wait()
        bs, ob = bufsets[cur], obufs[cur]

        def add_row(r, _, bs=bs, ob=ob):
            for col in range(D // LANES):
                sl = pl.ds(col * LANES, LANES)
                ob[r, sl] = bs[0][r, sl] + bs[1][r, sl] + bs[2][r, sl] + bs[3][r, sl]
            return 0

        lax.fori_loop(0, CHUNK, add_row, 0)
        wcps[cur] = pltpu.async_copy(
            ob, out.at[pl.ds(base + k * CHUNK, CHUNK)], wsems[cur])
    for cp in wcps:
        if cp is not None:
            cp.wait()


_mesh = plsc.VectorSubcoreMesh(core_axis_name="c", subcore_axis_name="s")

_sc_call = functools.partial(
    pl.kernel,
    mesh=_mesh,
    out_type=jax.ShapeDtypeStruct((N, D), jnp.float32),
    scratch_types=[
        pltpu.VMEM((LEVELS, NCHUNK, CHUNK), jnp.int32),
        pltpu.VMEM((CHUNK, D), jnp.float32),
        pltpu.VMEM((CHUNK, D), jnp.float32),
        pltpu.VMEM((CHUNK, D), jnp.float32),
        pltpu.VMEM((CHUNK, D), jnp.float32),
        pltpu.VMEM((CHUNK, D), jnp.float32),
        pltpu.VMEM((CHUNK, D), jnp.float32),
        pltpu.VMEM((CHUNK, D), jnp.float32),
        pltpu.VMEM((CHUNK, D), jnp.float32),
        pltpu.VMEM((CHUNK, D), jnp.float32),
        pltpu.VMEM((CHUNK, D), jnp.float32),
        pltpu.SemaphoreType.DMA,
        pltpu.SemaphoreType.DMA,
        pltpu.SemaphoreType.DMA,
        pltpu.SemaphoreType.DMA,
    ],
)(_body)


@jax.jit
def kernel(coords, emb0, emb1, emb2, emb3):
    coords_r = coords.T.reshape(LEVELS, NW, NCHUNK, CHUNK).transpose(1, 0, 2, 3)
    return _sc_call(coords_r, emb0, emb1, emb2, emb3)
